# Initial kernel scaffold; baseline (speedup 1.0000x reference)
import jax
import jax.numpy as jnp
from jax.experimental import pallas as pl


def kernel(x, weight):
    raise NotImplementedError("write your pallas kernel here")


# speedup vs baseline: 1.6844x; 1.6844x over previous
import functools

import jax
import jax.numpy as jnp
from jax import lax
from jax.experimental import pallas as pl
from jax.experimental.pallas import tpu as pltpu
from jax.experimental.pallas import tpu_sc as plsc

CHUNK = 128


def kernel(x, weight):
    batch, hist = x.shape
    vocab, dim = weight.shape
    total = batch * hist

    info = plsc.get_sparse_core_info()
    nw = info.num_cores * info.num_subcores
    per_w = total // nw
    nchunk = per_w // CHUNK
    assert per_w * nw == total and nchunk * CHUNK == per_w

    idx = x.reshape(total // CHUNK, CHUNK).astype(jnp.int32)

    mesh = plsc.VectorSubcoreMesh(core_axis_name="c", subcore_axis_name="s")

    @functools.partial(
        pl.kernel,
        out_type=jax.ShapeDtypeStruct((total, dim), jnp.float32),
        mesh=mesh,
        compiler_params=pltpu.CompilerParams(use_tc_tiling_on_sc=False),
        scratch_types=[
            pltpu.VMEM((nchunk, CHUNK), jnp.int32),
            pltpu.VMEM((CHUNK, dim), jnp.float32),
            pltpu.SemaphoreType.DMA,
        ],
    )
    def gather_kernel(idx_hbm, table_hbm, out_hbm, idx_v, rows_v, sem):
        wid = lax.axis_index("s") * info.num_cores + lax.axis_index("c")
        pltpu.sync_copy(idx_hbm.at[pl.ds(wid * nchunk, nchunk)], idx_v)

        def body(c, carry):
            pltpu.async_copy(table_hbm.at[idx_v.at[c]], rows_v, sem).wait()
            pltpu.sync_copy(
                rows_v, out_hbm.at[pl.ds((wid * nchunk + c) * CHUNK, CHUNK)]
            )
            return carry

        lax.fori_loop(0, nchunk, body, 0)

    out = gather_kernel(idx, weight)
    return out.reshape(batch, hist, dim)


# speedup vs baseline: 1.8602x; 1.1043x over previous
import functools

import jax
import jax.numpy as jnp
from jax import lax
from jax.experimental import pallas as pl
from jax.experimental.pallas import tpu as pltpu
from jax.experimental.pallas import tpu_sc as plsc

CHUNK = 128
K = 5


def kernel(x, weight):
    batch, hist = x.shape
    vocab, dim = weight.shape
    total = batch * hist

    info = plsc.get_sparse_core_info()
    nw = info.num_cores * info.num_subcores
    per_w = total // nw
    nchunk = per_w // CHUNK
    ngroups = nchunk // K
    assert per_w * nw == total and nchunk * CHUNK == per_w
    assert ngroups * K == nchunk and ngroups % 2 == 0

    idx = x.reshape(total // CHUNK, CHUNK).astype(jnp.int32)

    mesh = plsc.VectorSubcoreMesh(core_axis_name="c", subcore_axis_name="s")

    @functools.partial(
        pl.kernel,
        out_type=jax.ShapeDtypeStruct((total, dim), jnp.float32),
        mesh=mesh,
        compiler_params=pltpu.CompilerParams(use_tc_tiling_on_sc=False),
        scratch_types=[
            pltpu.VMEM((nchunk, CHUNK), jnp.int32),
            pltpu.VMEM((2, K * CHUNK, dim), jnp.float32),
            pltpu.SemaphoreType.DMA,
            pltpu.SemaphoreType.DMA,
            pltpu.SemaphoreType.DMA,
            pltpu.SemaphoreType.DMA,
        ],
    )
    def gather_kernel(
        idx_hbm, table_hbm, out_hbm, idx_v, rows_v, gsem0, gsem1, ssem0, ssem1
    ):
        wid = lax.axis_index("s") * info.num_cores + lax.axis_index("c")
        base_chunk = wid * nchunk
        pltpu.sync_copy(idx_hbm.at[pl.ds(base_chunk, nchunk)], idx_v)

        gsems = (gsem0, gsem1)
        ssems = (ssem0, ssem1)

        def fire_gathers(g, s):
            for k in range(K):
                pltpu.async_copy(
                    table_hbm.at[idx_v.at[g * K + k]],
                    rows_v.at[s, pl.ds(k * CHUNK, CHUNK)],
                    gsems[s],
                )

        def drain_gathers(s):
            for k in range(K):
                pltpu.make_async_copy(
                    table_hbm.at[pl.ds(0, CHUNK)],
                    rows_v.at[s, pl.ds(k * CHUNK, CHUNK)],
                    gsems[s],
                ).wait()

        def fire_store(g, s):
            pltpu.async_copy(
                rows_v.at[s],
                out_hbm.at[pl.ds((base_chunk + g * K) * CHUNK, K * CHUNK)],
                ssems[s],
            )

        def wait_store(s):
            pltpu.make_async_copy(
                table_hbm.at[pl.ds(0, K * CHUNK)],
                rows_v.at[s],
                ssems[s],
            ).wait()

        def body(g2, carry, last):
            a = 2 * g2
            drain_gathers(0)
            fire_store(a, 0)
            drain_gathers(1)
            fire_store(a + 1, 1)
            wait_store(0)
            if not last:
                fire_gathers(a + 2, 0)
            wait_store(1)
            if not last:
                fire_gathers(a + 3, 1)
            return carry

        fire_gathers(0, 0)
        fire_gathers(1, 1)
        lax.fori_loop(0, ngroups // 2 - 1, lambda g2, c: body(g2, c, False), 0)
        body(ngroups // 2 - 1, 0, True)

    out = gather_kernel(idx, weight)
    return out.reshape(batch, hist, dim)


# speedup vs baseline: 1.8673x; 1.0038x over previous
import functools

import jax
import jax.numpy as jnp
from jax import lax
from jax.experimental import pallas as pl
from jax.experimental.pallas import tpu as pltpu
from jax.experimental.pallas import tpu_sc as plsc

CHUNK = 512
K = 1


def kernel(x, weight):
    batch, hist = x.shape
    vocab, dim = weight.shape
    total = batch * hist

    info = plsc.get_sparse_core_info()
    nw = info.num_cores * info.num_subcores
    per_w = total // nw
    nchunk = per_w // CHUNK
    ngroups = nchunk // K
    assert per_w * nw == total and nchunk * CHUNK == per_w
    assert ngroups * K == nchunk and ngroups % 2 == 0

    idx = x.reshape(total // CHUNK, CHUNK).astype(jnp.int32)

    mesh = plsc.VectorSubcoreMesh(core_axis_name="c", subcore_axis_name="s")

    @functools.partial(
        pl.kernel,
        out_type=jax.ShapeDtypeStruct((total, dim), jnp.float32),
        mesh=mesh,
        compiler_params=pltpu.CompilerParams(use_tc_tiling_on_sc=False),
        scratch_types=[
            pltpu.VMEM((nchunk, CHUNK), jnp.int32),
            pltpu.VMEM((2, K * CHUNK, dim), jnp.float32),
            pltpu.SemaphoreType.DMA,
            pltpu.SemaphoreType.DMA,
            pltpu.SemaphoreType.DMA,
            pltpu.SemaphoreType.DMA,
        ],
    )
    def gather_kernel(
        idx_hbm, table_hbm, out_hbm, idx_v, rows_v, gsem0, gsem1, ssem0, ssem1
    ):
        wid = lax.axis_index("s") * info.num_cores + lax.axis_index("c")
        base_chunk = wid * nchunk
        pltpu.sync_copy(idx_hbm.at[pl.ds(base_chunk, nchunk)], idx_v)

        gsems = (gsem0, gsem1)
        ssems = (ssem0, ssem1)

        def fire_gathers(g, s):
            for k in range(K):
                pltpu.async_copy(
                    table_hbm.at[idx_v.at[g * K + k]],
                    rows_v.at[s, pl.ds(k * CHUNK, CHUNK)],
                    gsems[s],
                )

        def drain_gathers(s):
            for k in range(K):
                pltpu.make_async_copy(
                    table_hbm.at[pl.ds(0, CHUNK)],
                    rows_v.at[s, pl.ds(k * CHUNK, CHUNK)],
                    gsems[s],
                ).wait()

        def fire_store(g, s):
            pltpu.async_copy(
                rows_v.at[s],
                out_hbm.at[pl.ds((base_chunk + g * K) * CHUNK, K * CHUNK)],
                ssems[s],
            )

        def wait_store(s):
            pltpu.make_async_copy(
                table_hbm.at[pl.ds(0, K * CHUNK)],
                rows_v.at[s],
                ssems[s],
            ).wait()

        def body(g2, carry, last):
            a = 2 * g2
            drain_gathers(0)
            fire_store(a, 0)
            drain_gathers(1)
            fire_store(a + 1, 1)
            wait_store(0)
            if not last:
                fire_gathers(a + 2, 0)
            wait_store(1)
            if not last:
                fire_gathers(a + 3, 1)
            return carry

        fire_gathers(0, 0)
        fire_gathers(1, 1)
        lax.fori_loop(0, ngroups // 2 - 1, lambda g2, c: body(g2, c, False), 0)
        body(ngroups // 2 - 1, 0, True)

    out = gather_kernel(idx, weight)
    return out.reshape(batch, hist, dim)
